# baseline (speedup 1.0000x reference)
import jax
import jax.numpy as jnp
from jax.experimental import pallas as pl
from jax.experimental.pallas import tpu as pltpu

N = 1024
D = 256
H = 4
HD = H * D
NI = 4
NH = 1019
OUT = 1023
INV_SQRT_D = 1.0 / (D ** 0.5)


def _mm_t(a, b):
    return jax.lax.dot_general(a, b, (((1,), (1,)), ((), ())),
                               preferred_element_type=jnp.float32)


def _layer(x, wqp, wkp, wvp, wop, we, b, ew1, ew2, row, cmask, mask16, mask4):
    x4 = x[0:NI, :]
    xo = x[OUT:OUT + 1, :]
    cp, wk = wkp
    cp.wait()
    wkv = wk[:]
    k4 = _mm_t(x4, wkv)
    cp, wq = wqp
    cp.wait()
    wqv = wq[:]
    qo = _mm_t(xo, wqv)
    kb = jnp.where(mask16, jnp.concatenate([k4, k4, k4, k4], axis=0), 0.0)
    kbq = jnp.dot(kb, wqv, preferred_element_type=jnp.float32)
    l1 = _mm_t(kbq, x) * INV_SQRT_D
    l1 = l1 + jnp.concatenate(
        [ew1 * we[0, 0], ew1 * we[1, 0], ew1 * we[2, 0], ew1 * we[3, 0]],
        axis=0)
    qb = jnp.where(mask4, jnp.broadcast_to(qo, (H, HD)), 0.0)
    qbk = jnp.dot(qb, wkv, preferred_element_type=jnp.float32)
    l2 = _mm_t(qbk, x) * INV_SQRT_D
    l2 = l2 + jnp.concatenate(
        [ew2 * we[0, 0], ew2 * we[1, 0], ew2 * we[2, 0], ew2 * we[3, 0]],
        axis=0)
    l1 = jnp.where(l1 >= 0, l1, 0.2 * l1)
    l2 = jnp.where(l2 >= 0, l2, 0.2 * l2)
    neg = jnp.float32(-1e30)
    l1 = jnp.where(cmask, l1, neg)
    l2 = jnp.where(cmask, l2, neg)
    m_list = []
    for h in range(H):
        mh = jnp.maximum(jnp.max(l1[h * NI:(h + 1) * NI, :]),
                         jnp.max(l2[h:h + 1, :]))
        m_list.append(mh)
    m16 = jnp.concatenate(
        [jnp.broadcast_to(m, (NI, 1)) for m in m_list], axis=0)
    m4 = jnp.concatenate(
        [jnp.broadcast_to(m, (1, 1)) for m in m_list], axis=0)
    e1 = jnp.exp(l1 - m16)
    e2 = jnp.exp(l2 - m4)
    i_list = []
    for h in range(H):
        sh = jnp.sum(e1[h * NI:(h + 1) * NI, :]) + jnp.sum(e2[h:h + 1, :])
        i_list.append(1.0 / sh)
    a1 = e1 * jnp.concatenate(
        [jnp.broadcast_to(i, (NI, 1)) for i in i_list], axis=0)
    a2 = e2 * jnp.concatenate(
        [jnp.broadcast_to(i, (1, 1)) for i in i_list], axis=0)
    cp, wv = wvp
    cp.wait()
    wvv = wv[:]
    v4 = _mm_t(x4, wvv)
    vb = jnp.where(mask16, jnp.concatenate([v4, v4, v4, v4], axis=0), 0.0)
    cp, wo = wop
    cp.wait()
    wov = wo[:]
    u = jax.lax.dot_general(vb, wov, (((1,), (1,)), ((), ())),
                            preferred_element_type=jnp.float32)
    out = b + x + jax.lax.dot_general(
        a1, u, (((0,), (0,)), ((), ())),
        preferred_element_type=jnp.float32)
    a2x = jnp.dot(a2, x, preferred_element_type=jnp.float32)
    rov = _mm_t(a2x, wvv)
    ro = jnp.sum(jnp.where(mask4, rov, 0.0), axis=0, keepdims=True)
    roc = jax.lax.dot_general(ro, wov, (((1,), (1,)), ((), ())),
                              preferred_element_type=jnp.float32)
    out = jnp.where(row == OUT, out + roc, out)
    return out


def _gnn_kernel(xin_ref, x_ref, ew1_ref, ew2_ref,
                wq1_ref, wk1_ref, wv1_ref, we1_ref, wo1_ref, b1_ref,
                wq2_ref, wk2_ref, wv2_ref, we2_ref, wo2_ref, b2_ref,
                ow_ref, ob_ref,
                y_ref, xout_ref,
                wq1_v, wk1_v, wv1_v, wo1_v, wq2_v, wk2_v, wv2_v, wo2_v,
                sems):
    pairs = [(wk1_ref, wk1_v), (wq1_ref, wq1_v), (wv1_ref, wv1_v),
             (wo1_ref, wo1_v), (wk2_ref, wk2_v), (wq2_ref, wq2_v),
             (wv2_ref, wv2_v), (wo2_ref, wo2_v)]
    cps = []
    for i, (src, dst) in enumerate(pairs):
        cp = pltpu.make_async_copy(src, dst, sems.at[i])
        cp.start()
        cps.append((cp, dst))
    row = jax.lax.broadcasted_iota(jnp.int32, (N, 1), 0)
    col = jax.lax.broadcasted_iota(jnp.int32, (1, D), 1)
    coln = jax.lax.broadcasted_iota(jnp.int32, (1, N), 1)
    cmask = (coln >= NI) & (coln < OUT)
    colf = jax.lax.broadcasted_iota(jnp.int32, (1, HD), 1) // D
    mask16 = colf == (jax.lax.broadcasted_iota(jnp.int32, (16, 1), 0) // NI)
    mask4 = colf == jax.lax.broadcasted_iota(jnp.int32, (H, 1), 0)
    x = x_ref[:]
    xin = jnp.concatenate(
        [xin_ref[:], jnp.zeros((N - NI, 1), jnp.float32)], axis=0)
    x = jnp.where((row < NI) & (col == 0), xin, x)
    ew1 = jnp.concatenate(
        [jnp.zeros((NI, NI), jnp.float32), ew1_ref[:],
         jnp.zeros((NI, 1), jnp.float32)], axis=1)
    ew2 = jnp.concatenate(
        [jnp.zeros((1, NI), jnp.float32), ew2_ref[:],
         jnp.zeros((1, 1), jnp.float32)], axis=1)
    x = _layer(x, cps[1], cps[0], cps[2], cps[3], we1_ref[:], b1_ref[:],
               ew1, ew2, row, cmask, mask16, mask4)
    x = jnp.maximum(x, 0.0)
    x = _layer(x, cps[5], cps[4], cps[6], cps[7], we2_ref[:], b2_ref[:],
               ew1, ew2, row, cmask, mask16, mask4)
    x = jnp.maximum(x, 0.0)
    xout_ref[:] = x
    y = jnp.sum(x[OUT:OUT + 1, :] * ow_ref[:], axis=1,
                keepdims=True) + ob_ref[:]
    y_ref[:] = jax.nn.sigmoid(y)


def kernel(x_input, node_features, edge_weights, c1_Wq, c1_Wk, c1_Wv, c1_We,
           c1_Wout_w, c1_Wout_b, c2_Wq, c2_Wk, c2_Wv, c2_We, c2_Wout_w,
           c2_Wout_b, out_w, out_b, edge_index):
    xin = x_input.reshape(NI, 1)
    ew1 = edge_weights[:NI * NH, 0].reshape(NI, NH)
    ew2 = edge_weights[NI * NH:, 0].reshape(1, NH)
    vmem = pl.BlockSpec(memory_space=pltpu.MemorySpace.VMEM)
    hbm = pl.BlockSpec(memory_space=pl.ANY)
    y, x_out = pl.pallas_call(
        _gnn_kernel,
        out_shape=[
            jax.ShapeDtypeStruct((1, 1), jnp.float32),
            jax.ShapeDtypeStruct((N, D), jnp.float32),
        ],
        in_specs=[vmem, vmem, vmem, vmem,
                  hbm, hbm, hbm, vmem, hbm, vmem,
                  hbm, hbm, hbm, vmem, hbm, vmem,
                  vmem, vmem],
        scratch_shapes=[pltpu.VMEM((HD, D), jnp.float32)] * 3
        + [pltpu.VMEM((D, HD), jnp.float32)]
        + [pltpu.VMEM((HD, D), jnp.float32)] * 3
        + [pltpu.VMEM((D, HD), jnp.float32)]
        + [pltpu.SemaphoreType.DMA((8,))],
    )(xin, node_features, ew1, ew2,
      c1_Wq, c1_Wk, c1_Wv, c1_We, c1_Wout_w, c1_Wout_b.reshape(1, D),
      c2_Wq, c2_Wk, c2_Wv, c2_We, c2_Wout_w, c2_Wout_b.reshape(1, D),
      out_w, out_b.reshape(1, 1))
    return (y[0, 0], x_out)


# speedup vs baseline: 1.1791x; 1.1791x over previous
import jax
import jax.numpy as jnp
from jax.experimental import pallas as pl
from jax.experimental.pallas import tpu as pltpu

N = 1024
D = 256
H = 4
HD = H * D
NI = 4
NH = 1019
OUT = 1023
INV_SQRT_D = 1.0 / (D ** 0.5)


def _mm_t(a, b):
    return jax.lax.dot_general(a, b, (((1,), (1,)), ((), ())),
                               preferred_element_type=jnp.float32)


def _layer(x, wqp, wkp, wvp, wop, we_row, b, ew1, ew2, row, cmask,
           mask16, mask4):
    x4 = x[0:NI, :]
    xo = x[OUT:OUT + 1, :]
    cp, wk = wkp
    cp.wait()
    wkv = wk[:]
    k4 = _mm_t(x4, wkv)
    cp, wq = wqp
    cp.wait()
    wqv = wq[:]
    qo = _mm_t(xo, wqv)
    kb = jnp.where(mask16, jnp.concatenate([k4, k4, k4, k4], axis=0), 0.0)
    kbq = jnp.dot(kb, wqv, preferred_element_type=jnp.float32)
    l1 = _mm_t(kbq, x) * INV_SQRT_D
    l1 = l1 + jnp.concatenate(
        [ew1 * we_row[0, 0], ew1 * we_row[0, 1],
         ew1 * we_row[0, 2], ew1 * we_row[0, 3]], axis=0)
    qb = jnp.where(mask4, jnp.broadcast_to(qo, (H, HD)), 0.0)
    qbk = jnp.dot(qb, wkv, preferred_element_type=jnp.float32)
    l2 = _mm_t(qbk, x) * INV_SQRT_D
    l2 = l2 + jnp.concatenate(
        [ew2 * we_row[0, 0], ew2 * we_row[0, 1],
         ew2 * we_row[0, 2], ew2 * we_row[0, 3]], axis=0)
    l1 = jnp.where(l1 >= 0, l1, 0.2 * l1)
    l2 = jnp.where(l2 >= 0, l2, 0.2 * l2)
    neg = jnp.float32(-1e30)
    l1 = jnp.where(cmask, l1, neg)
    l2 = jnp.where(cmask, l2, neg)
    m_list = []
    for h in range(H):
        mh = jnp.maximum(jnp.max(l1[h * NI:(h + 1) * NI, :]),
                         jnp.max(l2[h:h + 1, :]))
        m_list.append(mh)
    m16 = jnp.concatenate(
        [jnp.broadcast_to(m, (NI, 1)) for m in m_list], axis=0)
    m4 = jnp.concatenate(
        [jnp.broadcast_to(m, (1, 1)) for m in m_list], axis=0)
    e1 = jnp.exp(l1 - m16)
    e2 = jnp.exp(l2 - m4)
    i_list = []
    for h in range(H):
        sh = jnp.sum(e1[h * NI:(h + 1) * NI, :]) + jnp.sum(e2[h:h + 1, :])
        i_list.append(1.0 / sh)
    a1 = e1 * jnp.concatenate(
        [jnp.broadcast_to(i, (NI, 1)) for i in i_list], axis=0)
    a2 = e2 * jnp.concatenate(
        [jnp.broadcast_to(i, (1, 1)) for i in i_list], axis=0)
    a2x = jnp.dot(a2, x, preferred_element_type=jnp.float32)
    cp, wv = wvp
    cp.wait()
    wvv = wv[:]
    v4 = _mm_t(x4, wvv)
    vb = jnp.where(mask16, jnp.concatenate([v4, v4, v4, v4], axis=0), 0.0)
    rov = _mm_t(a2x, wvv)
    ro = jnp.sum(jnp.where(mask4, rov, 0.0), axis=0, keepdims=True)
    cp, wo = wop
    cp.wait()
    wov = wo[:]
    u = jax.lax.dot_general(vb, wov, (((1,), (1,)), ((), ())),
                            preferred_element_type=jnp.float32)
    out = b + x + jax.lax.dot_general(
        a1, u, (((0,), (0,)), ((), ())),
        preferred_element_type=jnp.float32)
    roc = jax.lax.dot_general(ro, wov, (((1,), (1,)), ((), ())),
                              preferred_element_type=jnp.float32)
    out = jnp.where(row == OUT, out + roc, out)
    return out


def _gnn_kernel(xin_ref, x_ref, p_ref,
                wq1_ref, wk1_ref, wv1_ref, wo1_ref,
                wq2_ref, wk2_ref, wv2_ref, wo2_ref,
                y_ref, xout_ref, s1, s2, sems):
    pairs = [(wk1_ref, s1.at[0]), (wq1_ref, s1.at[1]), (wv1_ref, s1.at[2]),
             (wo1_ref, s2.at[0]), (wk2_ref, s1.at[3]), (wq2_ref, s1.at[4]),
             (wv2_ref, s1.at[5]), (wo2_ref, s2.at[1])]
    cps = []
    for i, (src, dst) in enumerate(pairs):
        cp = pltpu.make_async_copy(src, dst, sems.at[i])
        cp.start()
        cps.append((cp, dst))
    row = jax.lax.broadcasted_iota(jnp.int32, (N, 1), 0)
    col = jax.lax.broadcasted_iota(jnp.int32, (1, D), 1)
    coln = jax.lax.broadcasted_iota(jnp.int32, (1, N), 1)
    cmask = (coln >= NI) & (coln < OUT)
    colf = jax.lax.broadcasted_iota(jnp.int32, (1, HD), 1) // D
    mask16 = colf == (jax.lax.broadcasted_iota(jnp.int32, (16, 1), 0) // NI)
    mask4 = colf == jax.lax.broadcasted_iota(jnp.int32, (H, 1), 0)
    x = x_ref[:]
    xin = jnp.concatenate(
        [xin_ref[:], jnp.zeros((N - NI, 1), jnp.float32)], axis=0)
    x = jnp.where((row < NI) & (col == 0), xin, x)
    p = p_ref[:]
    ew1 = p[0:NI, :]
    ew2 = p[NI:NI + 1, :]
    b1 = p[5:6, 0:D]
    we1 = p[5:6, D:D + H]
    b2 = p[6:7, 0:D]
    we2 = p[6:7, D:D + H]
    ow = p[7:8, 0:D]
    ob = p[7, D + H]
    x = _layer(x, cps[1], cps[0], cps[2], cps[3], we1, b1,
               ew1, ew2, row, cmask, mask16, mask4)
    x = jnp.maximum(x, 0.0)
    x = _layer(x, cps[5], cps[4], cps[6], cps[7], we2, b2,
               ew1, ew2, row, cmask, mask16, mask4)
    x = jnp.maximum(x, 0.0)
    xout_ref[:] = x
    y = jnp.sum(x[OUT:OUT + 1, :] * ow, axis=1, keepdims=True) + ob
    y_ref[:] = jax.nn.sigmoid(y)


def kernel(x_input, node_features, edge_weights, c1_Wq, c1_Wk, c1_Wv, c1_We,
           c1_Wout_w, c1_Wout_b, c2_Wq, c2_Wk, c2_Wv, c2_We, c2_Wout_w,
           c2_Wout_b, out_w, out_b, edge_index):
    z = jnp.zeros((1, N - D - H - 1), jnp.float32)
    ew1 = edge_weights[:NI * NH, 0].reshape(NI, NH)
    ew2 = edge_weights[NI * NH:, 0].reshape(1, NH)
    zc4 = jnp.zeros((NI, NI), jnp.float32)
    zc1 = jnp.zeros((NI, 1), jnp.float32)
    packed = jnp.concatenate([
        jnp.concatenate([zc4, ew1, zc1], axis=1),
        jnp.concatenate([zc4[0:1], ew2, zc1[0:1]], axis=1),
        jnp.concatenate([c1_Wout_b.reshape(1, D), c1_We.reshape(1, H),
                         jnp.zeros((1, 1), jnp.float32), z], axis=1),
        jnp.concatenate([c2_Wout_b.reshape(1, D), c2_We.reshape(1, H),
                         jnp.zeros((1, 1), jnp.float32), z], axis=1),
        jnp.concatenate([out_w.reshape(1, D), jnp.zeros((1, H), jnp.float32),
                         out_b.reshape(1, 1), z], axis=1),
    ], axis=0)
    xin = x_input.reshape(NI, 1)
    vmem = pl.BlockSpec(memory_space=pltpu.MemorySpace.VMEM)
    hbm = pl.BlockSpec(memory_space=pl.ANY)
    y, x_out = pl.pallas_call(
        _gnn_kernel,
        out_shape=[
            jax.ShapeDtypeStruct((1, 1), jnp.float32),
            jax.ShapeDtypeStruct((N, D), jnp.float32),
        ],
        in_specs=[vmem, vmem, vmem,
                  hbm, hbm, hbm, hbm, hbm, hbm, hbm, hbm],
        scratch_shapes=[pltpu.VMEM((6, HD, D), jnp.float32),
                        pltpu.VMEM((2, D, HD), jnp.float32),
                        pltpu.SemaphoreType.DMA((8,))],
    )(xin, node_features, packed,
      c1_Wq, c1_Wk, c1_Wv, c1_Wout_w, c2_Wq, c2_Wk, c2_Wv, c2_Wout_w)
    return (y[0, 0], x_out)


# speedup vs baseline: 1.2217x; 1.0361x over previous
import jax
import jax.numpy as jnp
from jax.experimental import pallas as pl
from jax.experimental.pallas import tpu as pltpu

N = 1024
D = 256
H = 4
HD = H * D
NI = 4
NH = 1019
OUT = 1023
INV_SQRT_D = 1.0 / (D ** 0.5)


def _mm_t(a, b):
    return jax.lax.dot_general(a, b, (((1,), (1,)), ((), ())),
                               preferred_element_type=jnp.float32)


def _layer(x, wqp, wkp, wvp, wop, we_row, b, ew1, ew2, row, cmask,
           mask16, mask4):
    x4 = x[0:NI, :]
    xo = x[OUT:OUT + 1, :]
    cp, wk = wkp
    cp.wait()
    wkv = wk[:]
    k4 = _mm_t(x4, wkv)
    cp, wq = wqp
    cp.wait()
    wqv = wq[:]
    qo = _mm_t(xo, wqv)
    kb = jnp.where(mask16, jnp.concatenate([k4, k4, k4, k4], axis=0), 0.0)
    kbq = jnp.dot(kb, wqv, preferred_element_type=jnp.float32)
    l1 = _mm_t(kbq, x) * INV_SQRT_D
    l1 = l1 + jnp.concatenate(
        [ew1 * we_row[0, 0], ew1 * we_row[0, 1],
         ew1 * we_row[0, 2], ew1 * we_row[0, 3]], axis=0)
    qb = jnp.where(mask4, jnp.broadcast_to(qo, (H, HD)), 0.0)
    qbk = jnp.dot(qb, wkv, preferred_element_type=jnp.float32)
    l2 = _mm_t(qbk, x) * INV_SQRT_D
    l2 = l2 + jnp.concatenate(
        [ew2 * we_row[0, 0], ew2 * we_row[0, 1],
         ew2 * we_row[0, 2], ew2 * we_row[0, 3]], axis=0)
    l1 = jnp.where(l1 >= 0, l1, 0.2 * l1)
    l2 = jnp.where(l2 >= 0, l2, 0.2 * l2)
    neg = jnp.float32(-1e30)
    l1 = jnp.where(cmask, l1, neg)
    l2 = jnp.where(cmask, l2, neg)
    m_list = []
    for h in range(H):
        mh = jnp.maximum(jnp.max(l1[h * NI:(h + 1) * NI, :]),
                         jnp.max(l2[h:h + 1, :]))
        m_list.append(mh)
    m16 = jnp.concatenate(
        [jnp.broadcast_to(m, (NI, 1)) for m in m_list], axis=0)
    m4 = jnp.concatenate(
        [jnp.broadcast_to(m, (1, 1)) for m in m_list], axis=0)
    e1 = jnp.exp(l1 - m16)
    e2 = jnp.exp(l2 - m4)
    i_list = []
    for h in range(H):
        sh = jnp.sum(e1[h * NI:(h + 1) * NI, :]) + jnp.sum(e2[h:h + 1, :])
        i_list.append(1.0 / sh)
    a1 = e1 * jnp.concatenate(
        [jnp.broadcast_to(i, (NI, 1)) for i in i_list], axis=0)
    a2 = e2 * jnp.concatenate(
        [jnp.broadcast_to(i, (1, 1)) for i in i_list], axis=0)
    a2x = jnp.dot(a2, x, preferred_element_type=jnp.float32)
    cp, wv = wvp
    cp.wait()
    wvv = wv[:]
    v4 = _mm_t(x4, wvv)
    vb = jnp.where(mask16, jnp.concatenate([v4, v4, v4, v4], axis=0), 0.0)
    rov = _mm_t(a2x, wvv)
    ro = jnp.sum(jnp.where(mask4, rov, 0.0), axis=0, keepdims=True)
    cp, wo = wop
    cp.wait()
    wov = wo[:]
    u = jax.lax.dot_general(vb, wov, (((1,), (1,)), ((), ())),
                            preferred_element_type=jnp.float32)
    out = b + x + jax.lax.dot_general(
        a1, u, (((0,), (0,)), ((), ())),
        preferred_element_type=jnp.float32)
    roc = jax.lax.dot_general(ro, wov, (((1,), (1,)), ((), ())),
                              preferred_element_type=jnp.float32)
    out = jnp.where(row == OUT, out + roc, out)
    return out


def _gnn_kernel(x_ref, p_ref,
                wq1_ref, wk1_ref, wv1_ref, wo1_ref,
                wq2_ref, wk2_ref, wv2_ref, wo2_ref,
                y_ref, xout_ref, s1, s2, sems):
    pairs = [(wk1_ref, s1.at[0]), (wq1_ref, s1.at[1]), (wv1_ref, s1.at[2]),
             (wo1_ref, s2.at[0]), (wk2_ref, s1.at[3]), (wq2_ref, s1.at[4]),
             (wv2_ref, s1.at[5]), (wo2_ref, s2.at[1])]
    cps = []
    for i, (src, dst) in enumerate(pairs):
        cp = pltpu.make_async_copy(src, dst, sems.at[i])
        cp.start()
        cps.append((cp, dst))
    row = jax.lax.broadcasted_iota(jnp.int32, (N, 1), 0)
    col = jax.lax.broadcasted_iota(jnp.int32, (1, D), 1)
    coln = jax.lax.broadcasted_iota(jnp.int32, (1, N), 1)
    cmask = (coln >= NI) & (coln < OUT)
    colf = jax.lax.broadcasted_iota(jnp.int32, (1, HD), 1) // D
    mask16 = colf == (jax.lax.broadcasted_iota(jnp.int32, (16, 1), 0) // NI)
    mask4 = colf == jax.lax.broadcasted_iota(jnp.int32, (H, 1), 0)
    x = x_ref[:]
    i4 = (jax.lax.broadcasted_iota(jnp.int32, (NI, NI), 0)
          == jax.lax.broadcasted_iota(jnp.int32, (NI, NI), 1)
          ).astype(jnp.float32)
    xin4 = jax.lax.dot_general(i4, p_ref[7:8, 300:300 + NI],
                               (((1,), (1,)), ((), ())),
                               preferred_element_type=jnp.float32)
    xin = jnp.concatenate(
        [xin4, jnp.zeros((N - NI, 1), jnp.float32)], axis=0)
    x = jnp.where((row < NI) & (col == 0), xin, x)
    p = p_ref[:]
    ew1 = p[0:NI, :]
    ew2 = p[NI:NI + 1, :]
    b1 = p[5:6, 0:D]
    we1 = p[5:6, D:D + H]
    b2 = p[6:7, 0:D]
    we2 = p[6:7, D:D + H]
    ow = p[7:8, 0:D]
    ob = p[7, D + H]
    x = _layer(x, cps[1], cps[0], cps[2], cps[3], we1, b1,
               ew1, ew2, row, cmask, mask16, mask4)
    x = jnp.maximum(x, 0.0)
    x = _layer(x, cps[5], cps[4], cps[6], cps[7], we2, b2,
               ew1, ew2, row, cmask, mask16, mask4)
    x = jnp.maximum(x, 0.0)
    xout_ref[:] = x
    y = jnp.sum(x[OUT:OUT + 1, :] * ow, axis=1, keepdims=True) + ob
    y_ref[:] = jax.nn.sigmoid(y)


def kernel(x_input, node_features, edge_weights, c1_Wq, c1_Wk, c1_Wv, c1_We,
           c1_Wout_w, c1_Wout_b, c2_Wq, c2_Wk, c2_Wv, c2_We, c2_Wout_w,
           c2_Wout_b, out_w, out_b, edge_index):
    z = jnp.zeros((1, N - D - H - 1), jnp.float32)
    z2 = jnp.zeros((1, 300 - D - H - 1), jnp.float32)
    z3 = jnp.zeros((1, N - 300 - NI), jnp.float32)
    ew1 = edge_weights[:NI * NH, 0].reshape(NI, NH)
    ew2 = edge_weights[NI * NH:, 0].reshape(1, NH)
    zc4 = jnp.zeros((NI, NI), jnp.float32)
    zc1 = jnp.zeros((NI, 1), jnp.float32)
    packed = jnp.concatenate([
        jnp.concatenate([zc4, ew1, zc1], axis=1),
        jnp.concatenate([zc4[0:1], ew2, zc1[0:1]], axis=1),
        jnp.concatenate([c1_Wout_b.reshape(1, D), c1_We.reshape(1, H),
                         jnp.zeros((1, 1), jnp.float32), z], axis=1),
        jnp.concatenate([c2_Wout_b.reshape(1, D), c2_We.reshape(1, H),
                         jnp.zeros((1, 1), jnp.float32), z], axis=1),
        jnp.concatenate([out_w.reshape(1, D), jnp.zeros((1, H), jnp.float32),
                         out_b.reshape(1, 1), z2, x_input.reshape(1, NI),
                         z3], axis=1),
    ], axis=0)
    vmem = pl.BlockSpec(memory_space=pltpu.MemorySpace.VMEM)
    hbm = pl.BlockSpec(memory_space=pl.ANY)
    y, x_out = pl.pallas_call(
        _gnn_kernel,
        out_shape=[
            jax.ShapeDtypeStruct((1, 1), jnp.float32),
            jax.ShapeDtypeStruct((N, D), jnp.float32),
        ],
        in_specs=[vmem, vmem,
                  hbm, hbm, hbm, hbm, hbm, hbm, hbm, hbm],
        scratch_shapes=[pltpu.VMEM((6, HD, D), jnp.float32),
                        pltpu.VMEM((2, D, HD), jnp.float32),
                        pltpu.SemaphoreType.DMA((8,))],
    )(node_features, packed,
      c1_Wq, c1_Wk, c1_Wv, c1_Wout_w, c2_Wq, c2_Wk, c2_Wv, c2_Wout_w)
    return (y[0, 0], x_out)
